# baseline (speedup 1.0000x reference)
import functools

import jax
import jax.numpy as jnp
from jax import lax
from jax.experimental import pallas as pl
from jax.experimental.pallas import tpu as pltpu
from jax.experimental.pallas import tpu_sc as plsc

VOCAB = 100000
EMB = 64
HID = 128
SEQ = 200
BATCH = 1024

_NW = 32
_TOK = SEQ * BATCH
_BPW = _TOK // _NW
_CH = 128
_NCH = _BPW // _CH


def _flatten_idx(src):
    def body(s_ref, o_ref):
        o_ref[...] = s_ref[...].reshape(_TOK // 128, 128)

    return pl.pallas_call(
        body,
        grid=(1,),
        in_specs=[pl.BlockSpec((SEQ, BATCH), lambda i: (0, 0))],
        out_specs=pl.BlockSpec((_TOK // 128, 128), lambda i: (0, 0)),
        out_shape=jax.ShapeDtypeStruct((_TOK // 128, 128), jnp.int32),
    )(src)


def _pad_table(emb):
    rows = VOCAB // 5

    def body(s_ref, o_ref):
        o_ref[:, :EMB] = s_ref[...]

    return pl.pallas_call(
        body,
        grid=(5,),
        in_specs=[pl.BlockSpec((rows, EMB), lambda i: (i, 0))],
        out_specs=pl.BlockSpec((rows, 128), lambda i: (i, 0)),
        out_shape=jax.ShapeDtypeStruct((VOCAB, 128), jnp.float32),
    )(emb)


def _sc_gather(table, idx2d):
    mesh = plsc.VectorSubcoreMesh(core_axis_name="c", subcore_axis_name="s")

    @functools.partial(
        pl.kernel,
        mesh=mesh,
        out_type=jax.ShapeDtypeStruct((_TOK, 128), jnp.float32),
        scratch_types=[
            pltpu.VMEM((_NCH, _CH), jnp.int32),
            pltpu.VMEM((5, _CH, 128), jnp.float32),
            pltpu.SemaphoreType.DMA,
            pltpu.SemaphoreType.DMA,
            pltpu.SemaphoreType.DMA,
            pltpu.SemaphoreType.DMA,
            pltpu.SemaphoreType.DMA,
        ],
        compiler_params=pltpu.CompilerParams(use_tc_tiling_on_sc=False),
    )
    def gather_kernel(table_hbm, idx_hbm, out_hbm, idx_v, bufs, s0, s1, s2, s3, s4):
        wid = lax.axis_index("s") * 2 + lax.axis_index("c")
        base = wid * _BPW
        pltpu.sync_copy(idx_hbm.at[pl.ds(wid * _NCH, _NCH)], idx_v)
        sems = (s0, s1, s2, s3, s4)

        for b in range(5):
            pltpu.async_copy(table_hbm.at[idx_v.at[b]], bufs.at[b], sems[b])

        def body(g, carry):
            j0 = 5 * g
            for b in range(5):
                j = j0 + b
                pltpu.make_async_copy(
                    table_hbm.at[idx_v.at[0]], bufs.at[b], sems[b]).wait()
                pltpu.sync_copy(
                    bufs.at[b, slice(None), pl.ds(0, EMB)],
                    out_hbm.at[pl.ds(base + j * _CH, _CH), pl.ds(0, EMB)])

                @pl.when(j + 5 < _NCH)
                def _():
                    pltpu.async_copy(
                        table_hbm.at[idx_v.at[j + 5]], bufs.at[b], sems[b])
            return carry

        lax.fori_loop(0, _NCH // 5, body, None)

    return gather_kernel(table, idx2d)


_TPI = 8
_NIT = SEQ // _TPI


def _gru_body(xf_ref, xb_ref, wrz_f, whn_f, win_f, brz_f, bhn_f, bin_f,
              wrz_b, whn_b, win_b, brz_b, bhn_b, bin_b, out_ref, hf, hb):
    wb_f = (wrz_f, whn_f, win_f, brz_f, bhn_f, bin_f)
    wb_b = (wrz_b, whn_b, win_b, brz_b, bhn_b, bin_b)
    t = pl.program_id(0)

    @pl.when(t == 0)
    def _init():
        hf[...] = jnp.zeros((BATCH, HID), jnp.float32)
        hb[...] = jnp.zeros((BATCH, HID), jnp.float32)

    def step(x, h_ref, w_rz, wh_n, wi_n, b_rz, bh_n, bi_n):
        h = h_ref[...]
        h_bf = h.astype(jnp.bfloat16)
        x_bf = x.astype(jnp.bfloat16)
        xh = jnp.concatenate([h_bf, x_bf], axis=1)
        s_rz = jnp.dot(xh, w_rz[...],
                       preferred_element_type=jnp.float32) + b_rz[...]
        hn = jnp.dot(h_bf, wh_n[...],
                     preferred_element_type=jnp.float32) + bh_n[...]
        i_n = jnp.dot(x_bf, wi_n[...],
                      preferred_element_type=jnp.float32) + bi_n[...]
        t_rz = jnp.tanh(s_rz)
        r = 0.5 * t_rz[:, :HID] + 0.5
        z = 0.5 * t_rz[:, HID:] + 0.5
        n = jnp.tanh(i_n + r * hn)
        h_ref[...] = n + z * (h - n)

    for k in range(_TPI):
        step(xf_ref[pl.ds(k * BATCH, BATCH), :EMB], hf, *wb_f)
        step(xb_ref[pl.ds((_TPI - 1 - k) * BATCH, BATCH), :EMB], hb, *wb_b)

    @pl.when(t == _NIT - 1)
    def _out():
        out_ref[0] = hf[...]
        out_ref[1] = hb[...]


def _tc_gru(embedded, pf, pb):
    full = lambda a: pl.BlockSpec(a.shape, lambda t: (0,) * a.ndim)
    return pl.pallas_call(
        _gru_body,
        grid=(_NIT,),
        in_specs=[
            pl.BlockSpec((_TPI * BATCH, 128), lambda t: (t, 0)),
            pl.BlockSpec((_TPI * BATCH, 128), lambda t: (_NIT - 1 - t, 0)),
        ] + [full(a) for a in pf] + [full(a) for a in pb],
        out_specs=pl.BlockSpec((2, BATCH, HID), lambda t: (0, 0, 0)),
        out_shape=jax.ShapeDtypeStruct((2, BATCH, HID), jnp.float32),
        scratch_shapes=[
            pltpu.VMEM((BATCH, HID), jnp.float32),
            pltpu.VMEM((BATCH, HID), jnp.float32),
        ],
    )(embedded, embedded, *pf, *pb)


def _prep_weights(Wih, Whh, bih, bhh):
    bf = jnp.bfloat16
    WiT, WhT = Wih.T, Whh.T
    w_rz = (0.5 * jnp.concatenate(
        [WhT[:, :2 * HID], WiT[:, :2 * HID]], axis=0)).astype(bf)
    wh_n = WhT[:, 2 * HID:].astype(bf)
    wi_n = WiT[:, 2 * HID:].astype(bf)
    b_rz = (0.5 * (bih[:2 * HID] + bhh[:2 * HID])).reshape(1, -1)
    bh_n = bhh[2 * HID:].reshape(1, -1)
    bi_n = bih[2 * HID:].reshape(1, -1)
    return w_rz, wh_n, wi_n, b_rz, bh_n, bi_n


def kernel(src, emb, w_ih_f, w_hh_f, b_ih_f, b_hh_f, w_ih_b, w_hh_b, b_ih_b, b_hh_b):
    idx2d = _flatten_idx(src)
    embedded = _sc_gather(_pad_table(emb), idx2d)
    pf = _prep_weights(w_ih_f, w_hh_f, b_ih_f, b_hh_f)
    pb = _prep_weights(w_ih_b, w_hh_b, b_ih_b, b_hh_b)
    return _tc_gru(embedded, pf, pb)


# speedup vs baseline: 1.1048x; 1.1048x over previous
import functools

import jax
import jax.numpy as jnp
from jax import lax
from jax.experimental import pallas as pl
from jax.experimental.pallas import tpu as pltpu
from jax.experimental.pallas import tpu_sc as plsc

VOCAB = 100000
EMB = 64
HID = 128
SEQ = 200
BATCH = 1024

_NW = 32
_TOK = SEQ * BATCH
_BPW = _TOK // _NW
_CH = 128
_NCH = _BPW // _CH


def _flatten_idx(src):
    def body(s_ref, o_ref):
        o_ref[...] = s_ref[...].reshape(_TOK // 128, 128)

    return pl.pallas_call(
        body,
        grid=(1,),
        in_specs=[pl.BlockSpec((SEQ, BATCH), lambda i: (0, 0))],
        out_specs=pl.BlockSpec((_TOK // 128, 128), lambda i: (0, 0)),
        out_shape=jax.ShapeDtypeStruct((_TOK // 128, 128), jnp.int32),
    )(src)


def _sc_gather(table, idx2d):
    mesh = plsc.VectorSubcoreMesh(core_axis_name="c", subcore_axis_name="s")

    @functools.partial(
        pl.kernel,
        mesh=mesh,
        out_type=jax.ShapeDtypeStruct((_TOK, 128), jnp.float32),
        scratch_types=[
            pltpu.VMEM((_NCH, _CH), jnp.int32),
            pltpu.VMEM((5, _CH, EMB), jnp.float32),
            pltpu.SemaphoreType.DMA,
            pltpu.SemaphoreType.DMA,
            pltpu.SemaphoreType.DMA,
            pltpu.SemaphoreType.DMA,
            pltpu.SemaphoreType.DMA,
        ],
        compiler_params=pltpu.CompilerParams(use_tc_tiling_on_sc=False),
    )
    def gather_kernel(table_hbm, idx_hbm, out_hbm, idx_v, bufs, s0, s1, s2, s3, s4):
        wid = lax.axis_index("s") * 2 + lax.axis_index("c")
        base = wid * _BPW
        pltpu.sync_copy(idx_hbm.at[pl.ds(wid * _NCH, _NCH)], idx_v)
        sems = (s0, s1, s2, s3, s4)

        for b in range(5):
            pltpu.async_copy(table_hbm.at[idx_v.at[b]], bufs.at[b], sems[b])

        def body(g, carry):
            j0 = 5 * g
            for b in range(5):
                j = j0 + b
                pltpu.make_async_copy(
                    table_hbm.at[idx_v.at[0]], bufs.at[b], sems[b]).wait()
                pltpu.sync_copy(
                    bufs.at[b],
                    out_hbm.at[pl.ds(base + j * _CH, _CH), pl.ds(0, EMB)])

                @pl.when(j + 5 < _NCH)
                def _():
                    pltpu.async_copy(
                        table_hbm.at[idx_v.at[j + 5]], bufs.at[b], sems[b])
            return carry

        lax.fori_loop(0, _NCH // 5, body, None)

    return gather_kernel(table, idx2d)


_TPI = 10
_NIT = SEQ // _TPI


def _gru_body(xf_ref, xb_ref, wrz_f, whn_f, win_f, brz_f, bhn_f, bin_f,
              wrz_b, whn_b, win_b, brz_b, bhn_b, bin_b, out_ref, hf, hb):
    wb_f = (wrz_f, whn_f, win_f, brz_f, bhn_f, bin_f)
    wb_b = (wrz_b, whn_b, win_b, brz_b, bhn_b, bin_b)
    t = pl.program_id(0)

    @pl.when(t == 0)
    def _init():
        hf[...] = jnp.zeros((BATCH, HID), jnp.float32)
        hb[...] = jnp.zeros((BATCH, HID), jnp.float32)

    def step(x, h_ref, w_rz, wh_n, wi_n, b_rz, bh_n, bi_n):
        h = h_ref[...]
        h_bf = h.astype(jnp.bfloat16)
        x_bf = x.astype(jnp.bfloat16)
        xh = jnp.concatenate([h_bf, x_bf], axis=1)
        s_rz = jnp.dot(xh, w_rz[...],
                       preferred_element_type=jnp.float32) + b_rz[...]
        hn = jnp.dot(h_bf, wh_n[...],
                     preferred_element_type=jnp.float32) + bh_n[...]
        i_n = jnp.dot(x_bf, wi_n[...],
                      preferred_element_type=jnp.float32) + bi_n[...]
        t_rz = jnp.tanh(s_rz)
        r = 0.5 * t_rz[:, :HID] + 0.5
        z = 0.5 * t_rz[:, HID:] + 0.5
        n = jnp.tanh(i_n + r * hn)
        h_ref[...] = n + z * (h - n)

    for k in range(_TPI):
        step(xf_ref[pl.ds(k * BATCH, BATCH), :EMB], hf, *wb_f)
        step(xb_ref[pl.ds((_TPI - 1 - k) * BATCH, BATCH), :EMB], hb, *wb_b)

    @pl.when(t == _NIT - 1)
    def _out():
        out_ref[0] = hf[...]
        out_ref[1] = hb[...]


def _tc_gru(embedded, pf, pb):
    full = lambda a: pl.BlockSpec(a.shape, lambda t: (0,) * a.ndim)
    return pl.pallas_call(
        _gru_body,
        grid=(_NIT,),
        in_specs=[
            pl.BlockSpec((_TPI * BATCH, 128), lambda t: (t, 0)),
            pl.BlockSpec((_TPI * BATCH, 128), lambda t: (_NIT - 1 - t, 0)),
        ] + [full(a) for a in pf] + [full(a) for a in pb],
        out_specs=pl.BlockSpec((2, BATCH, HID), lambda t: (0, 0, 0)),
        out_shape=jax.ShapeDtypeStruct((2, BATCH, HID), jnp.float32),
        scratch_shapes=[
            pltpu.VMEM((BATCH, HID), jnp.float32),
            pltpu.VMEM((BATCH, HID), jnp.float32),
        ],
    )(embedded, embedded, *pf, *pb)


def _prep_weights(Wih, Whh, bih, bhh):
    bf = jnp.bfloat16
    WiT, WhT = Wih.T, Whh.T
    w_rz = (0.5 * jnp.concatenate(
        [WhT[:, :2 * HID], WiT[:, :2 * HID]], axis=0)).astype(bf)
    wh_n = WhT[:, 2 * HID:].astype(bf)
    wi_n = WiT[:, 2 * HID:].astype(bf)
    b_rz = (0.5 * (bih[:2 * HID] + bhh[:2 * HID])).reshape(1, -1)
    bh_n = bhh[2 * HID:].reshape(1, -1)
    bi_n = bih[2 * HID:].reshape(1, -1)
    return w_rz, wh_n, wi_n, b_rz, bh_n, bi_n


def kernel(src, emb, w_ih_f, w_hh_f, b_ih_f, b_hh_f, w_ih_b, w_hh_b, b_ih_b, b_hh_b):
    idx2d = _flatten_idx(src)
    embedded = _sc_gather(emb, idx2d)
    pf = _prep_weights(w_ih_f, w_hh_f, b_ih_f, b_hh_f)
    pb = _prep_weights(w_ih_b, w_hh_b, b_ih_b, b_hh_b)
    return _tc_gru(embedded, pf, pb)


# speedup vs baseline: 1.1078x; 1.0028x over previous
import functools

import jax
import jax.numpy as jnp
from jax import lax
from jax.experimental import pallas as pl
from jax.experimental.pallas import tpu as pltpu
from jax.experimental.pallas import tpu_sc as plsc

VOCAB = 100000
EMB = 64
HID = 128
SEQ = 200
BATCH = 1024

_NW = 32
_TOK = SEQ * BATCH
_BPW = _TOK // _NW
_CH = 128
_NCH = _BPW // _CH


def _flatten_idx(src):
    def body(s_ref, o_ref):
        o_ref[...] = s_ref[...].reshape(_TOK // 128, 128)

    return pl.pallas_call(
        body,
        grid=(1,),
        in_specs=[pl.BlockSpec((SEQ, BATCH), lambda i: (0, 0))],
        out_specs=pl.BlockSpec((_TOK // 128, 128), lambda i: (0, 0)),
        out_shape=jax.ShapeDtypeStruct((_TOK // 128, 128), jnp.int32),
    )(src)


def _sc_gather(table, idx2d):
    mesh = plsc.VectorSubcoreMesh(core_axis_name="c", subcore_axis_name="s")

    @functools.partial(
        pl.kernel,
        mesh=mesh,
        out_type=jax.ShapeDtypeStruct((_TOK, 128), jnp.float32),
        scratch_types=[
            pltpu.VMEM((_NCH, _CH), jnp.int32),
            pltpu.VMEM((5, _CH, EMB), jnp.float32),
            pltpu.SemaphoreType.DMA,
            pltpu.SemaphoreType.DMA,
            pltpu.SemaphoreType.DMA,
            pltpu.SemaphoreType.DMA,
            pltpu.SemaphoreType.DMA,
        ],
        compiler_params=pltpu.CompilerParams(use_tc_tiling_on_sc=False),
    )
    def gather_kernel(table_hbm, idx_hbm, out_hbm, idx_v, bufs, s0, s1, s2, s3, s4):
        wid = lax.axis_index("s") * 2 + lax.axis_index("c")
        base = wid * _BPW
        pltpu.sync_copy(idx_hbm.at[pl.ds(wid * _NCH, _NCH)], idx_v)
        sems = (s0, s1, s2, s3, s4)

        for b in range(5):
            pltpu.async_copy(table_hbm.at[idx_v.at[b]], bufs.at[b], sems[b])

        def body(g, carry):
            j0 = 5 * g
            for b in range(5):
                j = j0 + b
                pltpu.make_async_copy(
                    table_hbm.at[idx_v.at[0]], bufs.at[b], sems[b]).wait()
                pltpu.sync_copy(
                    bufs.at[b],
                    out_hbm.at[pl.ds(base + j * _CH, _CH), pl.ds(0, EMB)])

                @pl.when(j + 5 < _NCH)
                def _():
                    pltpu.async_copy(
                        table_hbm.at[idx_v.at[j + 5]], bufs.at[b], sems[b])
            return carry

        lax.fori_loop(0, _NCH // 5, body, None)

    return gather_kernel(table, idx2d)


_TPI = 8
_NIT = SEQ // _TPI


def _gru_body(xf_ref, xb_ref, wrz_f, whn_f, win_f, brz_f, bhn_f, bin_f,
              wrz_b, whn_b, win_b, brz_b, bhn_b, bin_b, out_ref, hf, hb):
    wb_f = (wrz_f, whn_f, win_f, brz_f, bhn_f, bin_f)
    wb_b = (wrz_b, whn_b, win_b, brz_b, bhn_b, bin_b)
    t = pl.program_id(0)

    @pl.when(t == 0)
    def _init():
        hf[...] = jnp.zeros((BATCH, HID), jnp.float32)
        hb[...] = jnp.zeros((BATCH, HID), jnp.float32)

    def step(x, h_ref, w_rz, wh_n, wi_n, b_rz, bh_n, bi_n):
        h = h_ref[...]
        h_bf = h.astype(jnp.bfloat16)
        x_bf = x.astype(jnp.bfloat16)
        xh = jnp.concatenate([h_bf, x_bf], axis=1)
        s_rz = jnp.dot(xh, w_rz[...],
                       preferred_element_type=jnp.float32) + b_rz[...]
        hn = jnp.dot(h_bf, wh_n[...],
                     preferred_element_type=jnp.float32) + bh_n[...]
        i_n = jnp.dot(x_bf, wi_n[...],
                      preferred_element_type=jnp.float32) + bi_n[...]
        t_rz = jnp.tanh(s_rz)
        r = 0.5 * t_rz[:, :HID] + 0.5
        z = 0.5 * t_rz[:, HID:] + 0.5
        n = jnp.tanh(i_n + r * hn)
        h_ref[...] = n + z * (h - n)

    for k in range(_TPI):
        step(xf_ref[pl.ds(k * BATCH, BATCH), :EMB], hf, *wb_f)
        step(xb_ref[pl.ds((_TPI - 1 - k) * BATCH, BATCH), :EMB], hb, *wb_b)

    @pl.when(t == _NIT - 1)
    def _out():
        out_ref[0] = hf[...]
        out_ref[1] = hb[...]


def _tc_gru(embedded, pf, pb):
    full = lambda a: pl.BlockSpec(a.shape, lambda t: (0,) * a.ndim)
    return pl.pallas_call(
        _gru_body,
        grid=(_NIT,),
        in_specs=[
            pl.BlockSpec((_TPI * BATCH, 128), lambda t: (t, 0)),
            pl.BlockSpec((_TPI * BATCH, 128), lambda t: (_NIT - 1 - t, 0)),
        ] + [full(a) for a in pf] + [full(a) for a in pb],
        out_specs=pl.BlockSpec((2, BATCH, HID), lambda t: (0, 0, 0)),
        out_shape=jax.ShapeDtypeStruct((2, BATCH, HID), jnp.float32),
        scratch_shapes=[
            pltpu.VMEM((BATCH, HID), jnp.float32),
            pltpu.VMEM((BATCH, HID), jnp.float32),
        ],
    )(embedded, embedded, *pf, *pb)


def _prep_weights(Wih, Whh, bih, bhh):
    bf = jnp.bfloat16
    WiT, WhT = Wih.T, Whh.T
    w_rz = (0.5 * jnp.concatenate(
        [WhT[:, :2 * HID], WiT[:, :2 * HID]], axis=0)).astype(bf)
    wh_n = WhT[:, 2 * HID:].astype(bf)
    wi_n = WiT[:, 2 * HID:].astype(bf)
    b_rz = (0.5 * (bih[:2 * HID] + bhh[:2 * HID])).reshape(1, -1)
    bh_n = bhh[2 * HID:].reshape(1, -1)
    bi_n = bih[2 * HID:].reshape(1, -1)
    return w_rz, wh_n, wi_n, b_rz, bh_n, bi_n


def kernel(src, emb, w_ih_f, w_hh_f, b_ih_f, b_hh_f, w_ih_b, w_hh_b, b_ih_b, b_hh_b):
    idx2d = _flatten_idx(src)
    embedded = _sc_gather(emb, idx2d)
    pf = _prep_weights(w_ih_f, w_hh_f, b_ih_f, b_hh_f)
    pb = _prep_weights(w_ih_b, w_hh_b, b_ih_b, b_hh_b)
    return _tc_gru(embedded, pf, pb)
